# Initial kernel scaffold; baseline (speedup 1.0000x reference)
import jax
import jax.numpy as jnp
from jax.experimental import pallas as pl


def kernel(pred, actual):
    raise NotImplementedError("write your pallas kernel here")


# speedup vs baseline: 20.6933x; 20.6933x over previous
import functools

import jax
import jax.numpy as jnp
from jax import lax
from jax.experimental import pallas as pl
from jax.experimental.pallas import tpu as pltpu
from jax.experimental.pallas import tpu_sc as plsc

_N = 4194304
_NW = 32
_PER_TILE = _N // _NW
_CHUNK = 8192
_EPS = 1e-06

_LEVELS = ((31, 21, 2048), (21, 10, 2048), (10, 0, 1024))


def _make_sc_pass(level):
    check_shift, shift, nb = _LEVELS[level]
    mesh = plsc.VectorSubcoreMesh(core_axis_name="c", subcore_axis_name="s")

    def body(kw_hbm, pre_hbm, hist_out, suma_out, kw_v, hist_v, pre_v, acc_v):
        cid = lax.axis_index("c")
        sid = lax.axis_index("s")
        wid = sid * 2 + cid
        base = wid * _PER_TILE
        lane = lax.iota(jnp.int32, 16)
        lane_base = lane * nb

        pltpu.sync_copy(pre_hbm, pre_v)
        pre = pre_v[...]

        def zero_body(j, c):
            hist_v[pl.ds(j * 16, 16)] = jnp.zeros((16,), jnp.int32)
            return c

        lax.fori_loop(0, nb, zero_body, 0)

        def chunk_body(c, acc):
            off = base + c * _CHUNK
            pltpu.sync_copy(kw_hbm.at[pl.ds(off, _CHUNK)], kw_v)

            def vbody(i, acc):
                kw = kw_v[pl.ds(i * 16, 16)]
                ai = lax.shift_right_logical(kw, 31)
                w = 20 - 19 * ai
                key = kw & jnp.int32(0x7FFFFFFF)
                match = lax.shift_right_logical(key, check_shift) == pre
                wm = jnp.where(match, w, 0)
                bucket = lax.shift_right_logical(key, shift) & (nb - 1)
                plsc.addupdate_scatter(hist_v, [lane_base + bucket], wm)
                return acc + ai

            return lax.fori_loop(0, _CHUNK // 16, vbody, acc)

        acc = lax.fori_loop(0, _PER_TILE // _CHUNK, chunk_body,
                            jnp.zeros((16,), jnp.int32))
        acc_v[...] = acc

        def red_body(j, c):
            s = hist_v[pl.ds(j * 16, 16)]
            for k in range(1, 16):
                s = s + hist_v[pl.ds(k * nb + j * 16, 16)]
            hist_v[pl.ds(j * 16, 16)] = s
            return c

        lax.fori_loop(0, nb // 16, red_body, 0)
        pltpu.sync_copy(hist_v.at[pl.ds(0, nb)], hist_out.at[wid])
        pltpu.sync_copy(acc_v, suma_out.at[wid])

    return functools.partial(
        pl.kernel,
        out_type=(jax.ShapeDtypeStruct((_NW, nb), jnp.int32),
                  jax.ShapeDtypeStruct((_NW, 16), jnp.int32)),
        mesh=mesh,
        compiler_params=pltpu.CompilerParams(needs_layout_passes=False),
        scratch_types=[
            pltpu.VMEM((_CHUNK,), jnp.int32),
            pltpu.VMEM((nb * 16,), jnp.int32),
            pltpu.VMEM((16,), jnp.int32),
            pltpu.VMEM((16,), jnp.int32),
        ],
    )(body)


_sc_pass_0 = _make_sc_pass(0)
_sc_pass_1 = _make_sc_pass(1)
_sc_pass_2 = _make_sc_pass(2)


def _tc_finish_body(t_ref, p_ref, a_ref, extra_ref, cnt_ref, mse_ref):
    t = t_ref[0]
    p = p_ref[...]
    p = jnp.minimum(jnp.maximum(p, _EPS), 1.0 - _EPS)
    a = a_ref[...]
    mask = (a == 0.0) & (p > t)
    extra = jnp.sum(jnp.where(mask, -jnp.log(1.0 - p + t), 0.0))
    c = jnp.sum(jnp.where(mask, 1.0, 0.0))
    d = p - a
    m = jnp.sum(d * d)

    @pl.when(pl.program_id(0) == 0)
    def _init():
        extra_ref[0, 0] = 0.0
        cnt_ref[0, 0] = 0.0
        mse_ref[0, 0] = 0.0

    extra_ref[0, 0] += extra
    cnt_ref[0, 0] += c
    mse_ref[0, 0] += m


_ROWS = 2048
_COLS = _N // _ROWS
_GRID = 16
_BLK = _ROWS // _GRID


def _tc_finish(t, p2d, a2d):
    return pl.pallas_call(
        _tc_finish_body,
        grid=(_GRID,),
        in_specs=[
            pl.BlockSpec(memory_space=pltpu.SMEM),
            pl.BlockSpec((_BLK, _COLS), lambda i: (i, 0)),
            pl.BlockSpec((_BLK, _COLS), lambda i: (i, 0)),
        ],
        out_specs=[
            pl.BlockSpec(memory_space=pltpu.SMEM),
            pl.BlockSpec(memory_space=pltpu.SMEM),
            pl.BlockSpec(memory_space=pltpu.SMEM),
        ],
        out_shape=[
            jax.ShapeDtypeStruct((1, 1), jnp.float32),
            jax.ShapeDtypeStruct((1, 1), jnp.float32),
            jax.ShapeDtypeStruct((1, 1), jnp.float32),
        ],
    )(t, p2d, a2d)


def _select_bucket(hist_rows, t_rem):
    h = jnp.sum(hist_rows, axis=0, dtype=jnp.int32)
    c_rev = jnp.cumsum(h[::-1], dtype=jnp.int32)[::-1]
    bstar = jnp.sum((c_rev.astype(jnp.float32) > t_rem).astype(jnp.int32)) - 1
    above = c_rev[bstar] - h[bstar]
    return bstar, above.astype(jnp.float32)


def kernel(pred, actual):
    p = jnp.minimum(jnp.maximum(pred, jnp.float32(_EPS)), jnp.float32(1.0 - _EPS))
    key = lax.bitcast_convert_type(p, jnp.int32)
    packed = jnp.where(actual > 0.0, key | jnp.int32(-2147483648), key)

    zeros16 = jnp.zeros((16,), jnp.int32)
    hist0, suma_rows = _sc_pass_0(packed, zeros16)
    suma = jnp.sum(suma_rows)
    total = 20 * (_N - suma) + suma
    t_target = total.astype(jnp.float32) * jnp.float32(0.04)

    b1, above1 = _select_bucket(hist0, t_target)
    t1 = t_target - above1

    hist1, _ = _sc_pass_1(packed, jnp.full((16,), b1, jnp.int32))
    b2, above2 = _select_bucket(hist1, t1)
    t2 = t1 - above2

    pre2 = (b1 << 11) | b2
    hist2, _ = _sc_pass_2(packed, jnp.full((16,), pre2, jnp.int32))
    b3, _ = _select_bucket(hist2, t2)

    k_star = (b1 << 21) | (b2 << 10) | b3
    threshold = lax.bitcast_convert_type(k_star.astype(jnp.int32), jnp.float32)

    p2d = pred.reshape(_ROWS, _COLS)
    a2d = actual.reshape(_ROWS, _COLS)
    extra, cnt, mse_sum = _tc_finish(threshold.reshape(1), p2d, a2d)

    mse = mse_sum[0, 0] / jnp.float32(_N)
    return mse + extra[0, 0] / cnt[0, 0]


# speedup vs baseline: 25.0298x; 1.2096x over previous
import functools

import jax
import jax.numpy as jnp
from jax import lax
from jax.experimental import pallas as pl
from jax.experimental.pallas import tpu as pltpu
from jax.experimental.pallas import tpu_sc as plsc

_N = 4194304
_NW = 32
_PER_TILE = _N // _NW
_CHUNK = 16384
_NCHUNK = _PER_TILE // _CHUNK
_PAIRS = _NCHUNK // 2
_UNROLL = 4
_EPS = 1e-06

_LEVELS = ((31, 21, 2048), (21, 10, 2048), (10, 0, 1024))


def _make_sc_pass(level):
    check_shift, shift, nb = _LEVELS[level]
    mesh = plsc.VectorSubcoreMesh(core_axis_name="c", subcore_axis_name="s")
    lvl0 = level == 0

    def body(kw_hbm, *rest):
        if lvl0:
            (hist_out, suma_out, kw_v0, kw_v1, hist_v, acc_v, sem0,
             sem1) = rest
            pre = None
        else:
            (pre_hbm, hist_out, kw_v0, kw_v1, hist_v, pre_v, sem0,
             sem1) = rest
        cid = lax.axis_index("c")
        sid = lax.axis_index("s")
        wid = sid * 2 + cid
        base = wid * _PER_TILE
        lane = lax.iota(jnp.int32, 16)
        lane_base = lane * nb

        if not lvl0:
            pltpu.sync_copy(pre_hbm, pre_v)
            pre = pre_v[...]

        zeros = jnp.zeros((16,), jnp.int32)

        def zero_body(j, c):
            for k in range(8):
                hist_v[pl.ds(j * 128 + k * 16, 16)] = zeros
            return c

        lax.fori_loop(0, nb // 8, zero_body, 0)

        def step(kw_v, i, acc):
            kw = kw_v[pl.ds(i * 16, 16)]
            ai = lax.shift_right_logical(kw, 31)
            w = 20 - 19 * ai
            key = kw & jnp.int32(0x7FFFFFFF)
            if lvl0:
                bucket = lax.shift_right_logical(key, shift)
                acc = acc + ai
            else:
                match = lax.shift_right_logical(key, check_shift) == pre
                w = jnp.where(match, w, 0)
                if shift:
                    bucket = lax.shift_right_logical(key, shift) & (nb - 1)
                else:
                    bucket = key & (nb - 1)
            plsc.addupdate_scatter(hist_v, [lane_base + bucket], w)
            return acc

        def process(kw_v, acc):
            def vbody(i, acc):
                for k in range(_UNROLL):
                    acc = step(kw_v, i * _UNROLL + k, acc)
                return acc

            return lax.fori_loop(0, _CHUNK // (16 * _UNROLL), vbody, acc)

        def start(c, buf, sem):
            pltpu.async_copy(kw_hbm.at[pl.ds(base + c * _CHUNK, _CHUNK)],
                             buf, sem)

        def wait(buf, sem):
            pltpu.make_async_copy(kw_hbm.at[pl.ds(0, _CHUNK)], buf,
                                  sem).wait()

        start(0, kw_v0, sem0)

        def pair_body(g, acc):
            start(2 * g + 1, kw_v1, sem1)
            wait(kw_v0, sem0)
            acc = process(kw_v0, acc)

            @pl.when(g < _PAIRS - 1)
            def _():
                start(2 * g + 2, kw_v0, sem0)

            wait(kw_v1, sem1)
            return process(kw_v1, acc)

        acc = lax.fori_loop(0, _PAIRS, pair_body, zeros)

        def red_body(j, c):
            s = hist_v[pl.ds(j * 16, 16)]
            for k in range(1, 16):
                s = s + hist_v[pl.ds(k * nb + j * 16, 16)]
            hist_v[pl.ds(j * 16, 16)] = s
            return c

        lax.fori_loop(0, nb // 16, red_body, 0)
        pltpu.sync_copy(hist_v.at[pl.ds(0, nb)], hist_out.at[wid])
        if lvl0:
            acc_v[...] = acc
            pltpu.sync_copy(acc_v, suma_out.at[wid])

    if lvl0:
        out_type = (jax.ShapeDtypeStruct((_NW, nb), jnp.int32),
                    jax.ShapeDtypeStruct((_NW, 16), jnp.int32))
        tail = [pltpu.VMEM((16,), jnp.int32)]
    else:
        out_type = jax.ShapeDtypeStruct((_NW, nb), jnp.int32)
        tail = [pltpu.VMEM((16,), jnp.int32)]
    return functools.partial(
        pl.kernel,
        out_type=out_type,
        mesh=mesh,
        compiler_params=pltpu.CompilerParams(needs_layout_passes=False),
        scratch_types=[
            pltpu.VMEM((_CHUNK,), jnp.int32),
            pltpu.VMEM((_CHUNK,), jnp.int32),
            pltpu.VMEM((nb * 16,), jnp.int32),
        ] + tail + [pltpu.SemaphoreType.DMA, pltpu.SemaphoreType.DMA],
    )(body)


_sc_pass_0 = _make_sc_pass(0)
_sc_pass_1 = _make_sc_pass(1)
_sc_pass_2 = _make_sc_pass(2)


def _tc_finish_body(t_ref, p_ref, a_ref, extra_ref, cnt_ref, mse_ref):
    t = t_ref[0]
    p = p_ref[...]
    p = jnp.minimum(jnp.maximum(p, _EPS), 1.0 - _EPS)
    a = a_ref[...]
    mask = (a == 0.0) & (p > t)
    extra = jnp.sum(jnp.where(mask, -jnp.log(1.0 - p + t), 0.0))
    c = jnp.sum(jnp.where(mask, 1.0, 0.0))
    d = p - a
    m = jnp.sum(d * d)

    @pl.when(pl.program_id(0) == 0)
    def _init():
        extra_ref[0, 0] = 0.0
        cnt_ref[0, 0] = 0.0
        mse_ref[0, 0] = 0.0

    extra_ref[0, 0] += extra
    cnt_ref[0, 0] += c
    mse_ref[0, 0] += m


_ROWS = 2048
_COLS = _N // _ROWS
_GRID = 16
_BLK = _ROWS // _GRID


def _tc_finish(t, p2d, a2d):
    return pl.pallas_call(
        _tc_finish_body,
        grid=(_GRID,),
        in_specs=[
            pl.BlockSpec(memory_space=pltpu.SMEM),
            pl.BlockSpec((_BLK, _COLS), lambda i: (i, 0)),
            pl.BlockSpec((_BLK, _COLS), lambda i: (i, 0)),
        ],
        out_specs=[
            pl.BlockSpec(memory_space=pltpu.SMEM),
            pl.BlockSpec(memory_space=pltpu.SMEM),
            pl.BlockSpec(memory_space=pltpu.SMEM),
        ],
        out_shape=[
            jax.ShapeDtypeStruct((1, 1), jnp.float32),
            jax.ShapeDtypeStruct((1, 1), jnp.float32),
            jax.ShapeDtypeStruct((1, 1), jnp.float32),
        ],
    )(t, p2d, a2d)


def _select_bucket(hist_rows, t_rem):
    h = jnp.sum(hist_rows, axis=0, dtype=jnp.int32)
    c_rev = jnp.cumsum(h[::-1], dtype=jnp.int32)[::-1]
    bstar = jnp.sum((c_rev.astype(jnp.float32) > t_rem).astype(jnp.int32)) - 1
    above = c_rev[bstar] - h[bstar]
    return bstar, above.astype(jnp.float32)


def kernel(pred, actual):
    p = jnp.minimum(jnp.maximum(pred, jnp.float32(_EPS)), jnp.float32(1.0 - _EPS))
    key = lax.bitcast_convert_type(p, jnp.int32)
    packed = jnp.where(actual > 0.0, key | jnp.int32(-2147483648), key)

    hist0, suma_rows = _sc_pass_0(packed)
    suma = jnp.sum(suma_rows)
    total = 20 * (_N - suma) + suma
    t_target = total.astype(jnp.float32) * jnp.float32(0.04)

    b1, above1 = _select_bucket(hist0, t_target)
    t1 = t_target - above1

    hist1 = _sc_pass_1(packed, jnp.full((16,), b1, jnp.int32))
    b2, above2 = _select_bucket(hist1, t1)
    t2 = t1 - above2

    pre2 = (b1 << 11) | b2
    hist2 = _sc_pass_2(packed, jnp.full((16,), pre2, jnp.int32))
    b3, _ = _select_bucket(hist2, t2)

    k_star = (b1 << 21) | (b2 << 10) | b3
    threshold = lax.bitcast_convert_type(k_star.astype(jnp.int32), jnp.float32)

    p2d = pred.reshape(_ROWS, _COLS)
    a2d = actual.reshape(_ROWS, _COLS)
    extra, cnt, mse_sum = _tc_finish(threshold.reshape(1), p2d, a2d)

    mse = mse_sum[0, 0] / jnp.float32(_N)
    return mse + extra[0, 0] / cnt[0, 0]


# speedup vs baseline: 55.4877x; 2.2169x over previous
import functools

import jax
import jax.numpy as jnp
from jax import lax
from jax.experimental import pallas as pl
from jax.experimental.pallas import tpu as pltpu
from jax.experimental.pallas import tpu_sc as plsc

_N = 4194304
_NW = 32
_PER_TILE = _N // _NW
_CHUNK = 16384
_NCHUNK = _PER_TILE // _CHUNK
_PAIRS = _NCHUNK // 2
_UNROLL = 16
_EPS = 1e-06

_NB = (1024, 2048, 1024)


def _make_sc_pass(level):
    nb = _NB[level]
    mesh = plsc.VectorSubcoreMesh(core_axis_name="c", subcore_axis_name="s")
    lvl0 = level == 0

    def body(*args):
        if lvl0:
            (kw_hbm, hist_out, kw_v0, kw_v1, hist_v, sem0, sem1) = args
            pre = None
        else:
            (kw_hbm, pre_hbm, hist_out, kw_v0, kw_v1, hist_v, pre_v, sem0,
             sem1) = args
        cid = lax.axis_index("c")
        sid = lax.axis_index("s")
        wid = sid * 2 + cid
        base = wid * _PER_TILE
        lane = lax.iota(jnp.int32, 16)
        lane_base = lane * nb

        if not lvl0:
            pltpu.sync_copy(pre_hbm, pre_v)
            pre = pre_v[...]

        zeros = jnp.zeros((16,), jnp.int32)
        w20 = jnp.full((16,), 20, jnp.int32)

        def zero_body(j, c):
            for k in range(8):
                hist_v[pl.ds(j * 128 + k * 16, 16)] = zeros
            return c

        lax.fori_loop(0, nb // 8, zero_body, 0)

        def process(kw_v):
            def vbody(i, c):
                kws = [kw_v[pl.ds((i * _UNROLL + k) * 16, 16)]
                       for k in range(_UNROLL)]
                if lvl0:
                    buckets = [lax.shift_right_logical(kw, 21) & 0x3FF
                               for kw in kws]
                    matches = [None] * _UNROLL
                    ws = [jnp.where(kw < 0, 1, w20) for kw in kws]
                elif level == 1:
                    checks = [lax.shift_right_logical(kw, 21) & 0x3FF
                              for kw in kws]
                    matches = [chk == pre for chk in checks]
                    buckets = [lax.shift_right_logical(kw, 10) & 0x7FF
                               for kw in kws]
                    ws = [jnp.where(kw < 0, 1, w20) for kw in kws]
                else:
                    checks = [lax.shift_right_logical(kw, 10) & 0x1FFFFF
                              for kw in kws]
                    matches = [chk == pre for chk in checks]
                    buckets = [kw & 0x3FF for kw in kws]
                    ws = [jnp.where(kw < 0, 1, w20) for kw in kws]
                idxs = [lane_base | b for b in buckets]
                for k in range(_UNROLL):
                    plsc.addupdate_scatter(hist_v, [idxs[k]], ws[k],
                                           mask=matches[k])
                return c

            return lax.fori_loop(0, _CHUNK // (16 * _UNROLL), vbody, 0)

        def start(c, buf, sem):
            pltpu.async_copy(kw_hbm.at[pl.ds(base + c * _CHUNK, _CHUNK)],
                             buf, sem)

        def wait(buf, sem):
            pltpu.make_async_copy(kw_hbm.at[pl.ds(0, _CHUNK)], buf,
                                  sem).wait()

        start(0, kw_v0, sem0)

        def pair_body(g, c):
            start(2 * g + 1, kw_v1, sem1)
            wait(kw_v0, sem0)
            process(kw_v0)

            @pl.when(g < _PAIRS - 1)
            def _():
                start(2 * g + 2, kw_v0, sem0)

            wait(kw_v1, sem1)
            process(kw_v1)
            return c

        lax.fori_loop(0, _PAIRS, pair_body, 0)

        def red_body(j, c):
            s = hist_v[pl.ds(j * 16, 16)]
            for k in range(1, 16):
                s = s + hist_v[pl.ds(k * nb + j * 16, 16)]
            hist_v[pl.ds(j * 16, 16)] = s
            return c

        lax.fori_loop(0, nb // 16, red_body, 0)
        pltpu.sync_copy(hist_v.at[pl.ds(0, nb)], hist_out.at[wid])

    scratch = [
        pltpu.VMEM((_CHUNK,), jnp.int32),
        pltpu.VMEM((_CHUNK,), jnp.int32),
        pltpu.VMEM((nb * 16,), jnp.int32),
    ] + ([] if lvl0 else [pltpu.VMEM((16,), jnp.int32)]) + [
        pltpu.SemaphoreType.DMA,
        pltpu.SemaphoreType.DMA,
    ]
    return functools.partial(
        pl.kernel,
        out_type=jax.ShapeDtypeStruct((_NW, nb), jnp.int32),
        mesh=mesh,
        compiler_params=pltpu.CompilerParams(needs_layout_passes=False),
        scratch_types=scratch,
    )(body)


_sc_pass_0 = _make_sc_pass(0)
_sc_pass_1 = _make_sc_pass(1)
_sc_pass_2 = _make_sc_pass(2)


def _tc_finish_body(t_ref, kw_ref, extra_ref, cnt_ref, mse_ref, res_ref):
    t = t_ref[0]
    kw = kw_ref[...]
    p = lax.bitcast_convert_type(kw & jnp.int32(0x7FFFFFFF), jnp.float32)
    azero = kw >= 0
    mask = azero & (p > t)
    extra = jnp.sum(jnp.where(mask, -jnp.log(1.0 - p + t), 0.0))
    c = jnp.sum(jnp.where(mask, 1.0, 0.0))
    d = jnp.where(azero, p, p - 1.0)
    m = jnp.sum(d * d)

    @pl.when(pl.program_id(0) == 0)
    def _init():
        extra_ref[0, 0] = 0.0
        cnt_ref[0, 0] = 0.0
        mse_ref[0, 0] = 0.0

    extra_ref[0, 0] += extra
    cnt_ref[0, 0] += c
    mse_ref[0, 0] += m

    @pl.when(pl.program_id(0) == _GRID - 1)
    def _fin():
        res_ref[0, 0] = (mse_ref[0, 0] / jnp.float32(_N)
                         + extra_ref[0, 0] / cnt_ref[0, 0])


_ROWS = 2048
_COLS = _N // _ROWS
_GRID = 16
_BLK = _ROWS // _GRID


def _tc_finish(t, kw2d):
    return pl.pallas_call(
        _tc_finish_body,
        grid=(_GRID,),
        in_specs=[
            pl.BlockSpec(memory_space=pltpu.SMEM),
            pl.BlockSpec((_BLK, _COLS), lambda i: (i, 0)),
        ],
        out_specs=[
            pl.BlockSpec(memory_space=pltpu.SMEM),
            pl.BlockSpec(memory_space=pltpu.SMEM),
            pl.BlockSpec(memory_space=pltpu.SMEM),
            pl.BlockSpec(memory_space=pltpu.SMEM),
        ],
        out_shape=[
            jax.ShapeDtypeStruct((1, 1), jnp.float32),
            jax.ShapeDtypeStruct((1, 1), jnp.float32),
            jax.ShapeDtypeStruct((1, 1), jnp.float32),
            jax.ShapeDtypeStruct((1, 1), jnp.float32),
        ],
    )(t, kw2d)


def _select_bucket(hist_rows, t_rem):
    h = jnp.sum(hist_rows, axis=0, dtype=jnp.int32)
    c_rev = jnp.cumsum(h[::-1], dtype=jnp.int32)[::-1]
    bstar = jnp.sum((c_rev.astype(jnp.float32) > t_rem).astype(jnp.int32)) - 1
    above = c_rev[bstar] - h[bstar]
    return bstar, above.astype(jnp.float32)


def kernel(pred, actual):
    p = jnp.minimum(jnp.maximum(pred, jnp.float32(_EPS)), jnp.float32(1.0 - _EPS))
    key = lax.bitcast_convert_type(p, jnp.int32)
    packed = jnp.where(actual > 0.0, key | jnp.int32(-2147483648), key)
    kw2d = packed.reshape(_ROWS, _COLS)

    hist0 = _sc_pass_0(packed)
    t_target = jnp.sum(hist0).astype(jnp.float32) * jnp.float32(0.04)

    b1, above1 = _select_bucket(hist0, t_target)
    t1 = t_target - above1

    hist1 = _sc_pass_1(packed, jnp.full((16,), b1, jnp.int32))
    b2, above2 = _select_bucket(hist1, t1)
    t2 = t1 - above2

    pre2 = (b1 << 11) | b2
    hist2 = _sc_pass_2(packed, jnp.full((16,), pre2, jnp.int32))
    b3, _ = _select_bucket(hist2, t2)

    k_star = (b1 << 21) | (b2 << 10) | b3
    threshold = lax.bitcast_convert_type(k_star.astype(jnp.int32), jnp.float32)

    _, _, _, res = _tc_finish(threshold.reshape(1), kw2d)
    return res[0, 0]
